# baseline (speedup 1.0000x reference)
import functools

import jax
import jax.numpy as jnp
from jax import lax
from jax.experimental import pallas as pl
from jax.experimental.pallas import tpu as pltpu
from jax.experimental.pallas import tpu_sc as plsc

B = 256; S = 64; H = 512; D = 64; NC = 512; SKEL = 256; K = 9
BN_EPS = 1e-5
L4 = S // 4


def _mm_body(x_ref, w_ref, s_ref, t_ref, o_ref, *, act):
    acc = jnp.dot(x_ref[...], w_ref[...], preferred_element_type=jnp.float32)
    y = acc * s_ref[...] + t_ref[...]
    if act == "lrelu":
        y = jnp.where(y >= 0, y, 0.2 * y)
    o_ref[...] = y


def _mm(x, w, scale, shift, act="none", mb=256, half=False):
    if half:
        x = x.astype(jnp.bfloat16)
        w = w.astype(jnp.bfloat16)
    M, Kd = x.shape
    N = w.shape[1]
    return pl.pallas_call(
        functools.partial(_mm_body, act=act),
        grid=(M // mb,),
        in_specs=[
            pl.BlockSpec((mb, Kd), lambda i: (i, 0)),
            pl.BlockSpec((Kd, N), lambda i: (0, 0)),
            pl.BlockSpec((1, N), lambda i: (0, 0)),
            pl.BlockSpec((1, N), lambda i: (0, 0)),
        ],
        out_specs=pl.BlockSpec((mb, N), lambda i: (i, 0)),
        out_shape=jax.ShapeDtypeStruct((M, N), jnp.float32),
    )(x, w, scale, shift)


def _enc_scan_body(gi_ref, h0_ref, whhT_ref, bhh_ref,
                   so_ref, to_ref, ys_ref, h_s):
    t = pl.program_id(0)

    @pl.when(t == 0)
    def _():
        h_s[...] = h0_ref[...]

    h = h_s[...]
    gh = jnp.dot(h, whhT_ref[...], preferred_element_type=jnp.float32)
    gh = gh + bhh_ref[...]
    gi = gi_ref[0]
    r = jax.nn.sigmoid(gi[:, :H] + gh[:, :H])
    z = jax.nn.sigmoid(gi[:, H:2 * H] + gh[:, H:2 * H])
    n = jnp.tanh(gi[:, 2 * H:] + r * gh[:, 2 * H:])
    h2 = (1.0 - z) * n + z * h
    h_s[...] = h2
    ys_ref[0] = so_ref[...] * h2 + to_ref[...]


def _enc_scan(gi, h0, whhT, bhh, so, to, reverse):
    if reverse:
        tmap = lambda t: (S - 1 - t, 0, 0)
    else:
        tmap = lambda t: (t, 0, 0)
    zmap2 = lambda t: (0, 0)
    return pl.pallas_call(
        _enc_scan_body,
        grid=(S,),
        in_specs=[
            pl.BlockSpec((1, B, 3 * H), tmap),
            pl.BlockSpec((B, H), zmap2),
            pl.BlockSpec((H, 3 * H), zmap2),
            pl.BlockSpec((1, 3 * H), zmap2),
            pl.BlockSpec((1, H), zmap2),
            pl.BlockSpec((1, H), zmap2),
        ],
        out_specs=pl.BlockSpec((1, B, H), tmap),
        out_shape=jax.ShapeDtypeStruct((S, B, H), jnp.float32),
        scratch_shapes=[pltpu.VMEM((B, H), jnp.float32)],
    )(gi, h0, whhT, bhh, so, to)


def _dec_scan_body(cur_ref, np_ref, u_ref, h0_ref, wrest_ref, bih1_ref,
                   woutT_ref,
                   g1whhT_ref, bhh1_ref, g2wihT_ref, bih2_ref, g2whhT_ref,
                   bhh2_ref, nfcT_ref, nfcb_ref, gate_ref,
                   rec_ref, hx0_s, hx1_s, outb_s):
    t = pl.program_id(0)

    @pl.when(t == 0)
    def _():
        hx0_s[...] = h0_ref[...]
        outb_s[...] = jnp.zeros_like(outb_s)

    gate = gate_ref[0, 0] > 0.0
    out = jnp.where(gate, np_ref[0], outb_s[...])
    gi1 = jnp.dot(cur_ref[0], wrest_ref[...],
                  preferred_element_type=jnp.float32) + bih1_ref[...]
    gi1 = gi1 + jnp.dot(out.astype(jnp.bfloat16), woutT_ref[...],
                        preferred_element_type=jnp.float32)
    h0p = hx0_s[...]
    gh1 = jnp.dot(h0p.astype(jnp.bfloat16), g1whhT_ref[...],
                  preferred_element_type=jnp.float32)
    gh1 = gh1 + bhh1_ref[...]
    r1 = jax.nn.sigmoid(gi1[:, :H] + gh1[:, :H])
    z1 = jax.nn.sigmoid(gi1[:, H:2 * H] + gh1[:, H:2 * H])
    n1 = jnp.tanh(gi1[:, 2 * H:] + r1 * gh1[:, 2 * H:])
    hx0n = (1.0 - z1) * n1 + z1 * h0p

    h1p = jnp.where(t == 0, hx0n, hx1_s[...])
    gi2 = jnp.dot(hx0n.astype(jnp.bfloat16), g2wihT_ref[...],
                  preferred_element_type=jnp.float32)
    gi2 = gi2 + bih2_ref[...]
    gh2 = jnp.dot(h1p.astype(jnp.bfloat16), g2whhT_ref[...],
                  preferred_element_type=jnp.float32)
    gh2 = gh2 + bhh2_ref[...]
    r2 = jax.nn.sigmoid(gi2[:, :H] + gh2[:, :H])
    z2 = jax.nn.sigmoid(gi2[:, H:2 * H] + gh2[:, H:2 * H])
    n2 = jnp.tanh(gi2[:, 2 * H:] + r2 * gh2[:, 2 * H:])
    hx1n = (1.0 - z2) * n2 + z2 * h1p

    no = jnp.dot(hx1n, nfcT_ref[...], preferred_element_type=jnp.float32)
    no = no + nfcb_ref[...]
    rec_ref[0] = no
    outb_s[...] = (jax.nn.sigmoid(no) - u_ref[0] > 0).astype(jnp.float32)
    hx0_s[...] = hx0n
    hx1_s[...] = hx1n


def _dec_scan(cur_rest, note_prev, u, h0, wrest, bih1, woutT, g1whhT, bhh1,
              g2wihT, bih2, g2whhT, bhh2, nfcT, nfcb, gate):
    tmap3 = lambda t: (t, 0, 0)
    zmap2 = lambda t: (0, 0)
    cw = cur_rest.shape[2]
    return pl.pallas_call(
        _dec_scan_body,
        grid=(S,),
        in_specs=[
            pl.BlockSpec((1, B, cw), tmap3),
            pl.BlockSpec((1, B, K), tmap3),
            pl.BlockSpec((1, B, K), tmap3),
            pl.BlockSpec((B, H), zmap2),
            pl.BlockSpec((cw, 3 * H), zmap2),
            pl.BlockSpec((1, 3 * H), zmap2),
            pl.BlockSpec((K, 3 * H), zmap2),
            pl.BlockSpec((H, 3 * H), zmap2),
            pl.BlockSpec((1, 3 * H), zmap2),
            pl.BlockSpec((H, 3 * H), zmap2),
            pl.BlockSpec((1, 3 * H), zmap2),
            pl.BlockSpec((H, 3 * H), zmap2),
            pl.BlockSpec((1, 3 * H), zmap2),
            pl.BlockSpec((H, K), zmap2),
            pl.BlockSpec((1, K), zmap2),
            pl.BlockSpec(memory_space=pltpu.SMEM),
        ],
        out_specs=pl.BlockSpec((1, B, K), tmap3),
        out_shape=jax.ShapeDtypeStruct((S, B, K), jnp.float32),
        scratch_shapes=[pltpu.VMEM((B, H), jnp.float32),
                        pltpu.VMEM((B, H), jnp.float32),
                        pltpu.VMEM((B, K), jnp.float32)],
    )(cur_rest, note_prev, u, h0, wrest, bih1, woutT, g1whhT, bhh1, g2wihT,
      bih2, g2whhT, bhh2, nfcT, nfcb, gate)


def _vq_argmin_body(f_ref, et_ref, idx_ref):
    f = f_ref[...]
    et = et_ref[...]
    f2 = jnp.sum(f * f, axis=1, keepdims=True)
    e2 = jnp.sum(et * et, axis=0, keepdims=True)
    dist = f2 - 2.0 * jnp.dot(f, et, preferred_element_type=jnp.float32) + e2
    idx_ref[...] = jnp.argmin(dist, axis=1).astype(jnp.int32)[:, None]


def _vq_argmin(flat, embT):
    M = flat.shape[0]
    return pl.pallas_call(
        _vq_argmin_body,
        in_specs=[pl.BlockSpec((M, D), lambda: (0, 0)),
                  pl.BlockSpec((D, NC), lambda: (0, 0))],
        out_specs=pl.BlockSpec((M, 1), lambda: (0, 0)),
        out_shape=jax.ShapeDtypeStruct((M, 1), jnp.int32),
    )(flat, embT)


def _sc_quant_gather(embed, idx):
    M = idx.shape[0]
    dp = 128
    table = jnp.pad(embed, ((0, 0), (0, dp - D)))
    info = plsc.get_sparse_core_info()
    nw = info.num_cores * info.num_subcores
    bpw = M // nw
    mesh = plsc.VectorSubcoreMesh(core_axis_name="c", subcore_axis_name="s")

    @functools.partial(
        pl.kernel, mesh=mesh,
        out_type=jax.ShapeDtypeStruct((M, dp), jnp.float32),
        scratch_types=[pltpu.VMEM((bpw,), jnp.int32),
                       pltpu.VMEM((bpw, dp), jnp.float32),
                       pltpu.SemaphoreType.DMA],
    )
    def gather_k(table_hbm, idx_hbm, out_hbm, idx_v, rows_v, sem):
        wid = lax.axis_index("s") * info.num_cores + lax.axis_index("c")
        base = wid * bpw
        pltpu.sync_copy(idx_hbm.at[pl.ds(base, bpw)], idx_v)
        pltpu.async_copy(table_hbm.at[idx_v], rows_v, sem).wait()
        pltpu.sync_copy(rows_v, out_hbm.at[pl.ds(base, bpw)])

    return gather_k(table, idx)[:, :D]


def _vq_loss_body(f_ref, q_ref, idx_ref, loss_ref, cmt_ref, perp_ref):
    dq = q_ref[...] - f_ref[...]
    mse = jnp.mean(dq * dq)
    cmt_ref[...] = mse.reshape(1, 1)
    loss_ref[...] = (mse + 0.2 * mse).reshape(1, 1)
    M = f_ref.shape[0]
    onehot = (idx_ref[...] ==
              lax.broadcasted_iota(jnp.int32, (M, NC), 1)).astype(jnp.float32)
    avg = jnp.sum(onehot, axis=0) * (1.0 / M)
    perp = jnp.exp(-jnp.sum(avg * jnp.log(avg + 1e-10)))
    perp_ref[...] = perp.reshape(1, 1)


def _vq_loss(flat, quant, idx):
    M = flat.shape[0]
    zmap = lambda: (0, 0)
    return pl.pallas_call(
        _vq_loss_body,
        in_specs=[pl.BlockSpec((M, D), zmap),
                  pl.BlockSpec((M, D), zmap),
                  pl.BlockSpec((M, 1), zmap)],
        out_specs=[pl.BlockSpec((1, 1), zmap)] * 3,
        out_shape=[jax.ShapeDtypeStruct((1, 1), jnp.float32)] * 3,
    )(flat, quant, idx)


def kernel(note, skel, skel_encoded, training, teacher_forcing_ratio, params):
    p = params
    f32 = jnp.float32
    inv = 1.0 / jnp.sqrt(jnp.asarray(1.0 + BN_EPS, f32))
    ones_h = jnp.ones((1, H), f32)

    note_t = note.transpose(1, 0, 2)
    skel_t = skel.transpose(1, 0, 2)

    x = _mm(note_t.reshape(S * B, K), p['enc_in_w'].T,
            jnp.ones((1, H), f32), p['enc_in_b'][None])
    gi_f = _mm(x, p['enc_gru_wih_f'].T, jnp.ones((1, 3 * H), f32),
               p['enc_gru_bih_f'][None]).reshape(S, B, 3 * H)
    gi_b = _mm(x, p['enc_gru_wih_b'].T, jnp.ones((1, 3 * H), f32),
               p['enc_gru_bih_b'][None]).reshape(S, B, 3 * H)

    w_hid = jnp.concatenate([p['enc_hfc_w'].T, p['dec_hfc_w'].T], axis=1)
    b_hid = jnp.concatenate([p['enc_hfc_b'], p['dec_hfc_b']])[None]
    hid_all = _mm(skel_encoded, w_hid, jnp.ones((1, 2 * H), f32), b_hid)
    h0_enc = hid_all[:, :H]
    h0_dec = hid_all[:, H:]

    sa = (p['aux1_g'] * inv)[None]
    ta = p['aux1_b'][None]
    yf = _enc_scan(gi_f, h0_enc, p['enc_gru_whh_f'].T,
                   p['enc_gru_bhh_f'][None], sa[:, :H], ta[:, :H], False)
    yb = _enc_scan(gi_b, h0_enc, p['enc_gru_whh_b'].T,
                   p['enc_gru_bhh_b'][None], sa[:, H:], ta[:, H:], True)
    y_seq = jnp.concatenate([yf, yb], axis=2)

    yp = jnp.pad(y_seq, ((1, 1), (0, 0), (0, 0)))
    x1 = jnp.concatenate([yp[j:j + S:2] for j in range(4)], axis=-1)
    w1 = p['c1_w'].transpose(2, 1, 0).reshape(8 * H, H)
    s1 = (p['bn1_g'] * inv)[None]
    t1 = (p['c1_b'] * s1[0] + p['bn1_b'])[None]
    h1 = _mm(x1.reshape(32 * B, 8 * H), w1, s1, t1, act="lrelu")

    hp = jnp.pad(h1.reshape(32, B, H), ((1, 1), (0, 0), (0, 0)))
    x2 = jnp.concatenate([hp[j:j + 31:2] for j in range(4)], axis=-1)
    w2 = p['c2_w'].transpose(2, 1, 0).reshape(4 * H, H)
    s2 = (p['bn2_g'] * inv)[None]
    t2 = (p['c2_b'] * s2[0] + p['bn2_b'])[None]
    h2 = _mm(x2.reshape(L4 * B, 4 * H), w2, s2, t2, act="lrelu")

    flat = _mm(h2, p['c3_w'][:, :, 0].T, jnp.ones((1, D), f32),
               p['c3_b'][None])

    idx2 = _vq_argmin(flat, p['embed'].T)
    idx = idx2.reshape(L4 * B)
    quant = _sc_quant_gather(p['embed'], idx)
    loss2, cmt2, perp2 = _vq_loss(flat, quant, idx2)

    q_sb = quant.reshape(L4, B, D)
    qz = q_sb.transpose(1, 2, 0)
    enc_idx = idx.reshape(L4, B).T[:, :, None]

    qp = jnp.pad(q_sb.astype(jnp.bfloat16), ((1, 2), (0, 0), (0, 0)))
    x1p = jnp.concatenate([qp[1:19], qp[0:18]], axis=-1)
    w_ct1 = jnp.concatenate(
        [jnp.concatenate([p['ct1_w'][:, :, 0], p['ct1_w'][:, :, 2]], axis=0),
         jnp.concatenate([p['ct1_w'][:, :, 1], p['ct1_w'][:, :, 3]], axis=0)],
        axis=1)
    sd1 = p['dbn1_g'] * inv
    td1 = p['ct1_b'] * sd1 + p['dbn1_b']
    o1 = _mm(x1p.reshape(18 * B, 2 * D), w_ct1,
             jnp.tile(sd1, 2)[None], jnp.tile(td1, 2)[None], act="lrelu",
             half=True)
    d1 = (o1.reshape(18, B, 2, H).transpose(0, 2, 1, 3)
          .reshape(36, B, H)[:35])

    d1p = jnp.pad(d1.astype(jnp.bfloat16), ((1, 0), (0, 0), (0, 0)))
    x2p = jnp.concatenate([d1p[1:33], d1p[0:32]], axis=-1)
    w_ct2 = jnp.concatenate(
        [jnp.concatenate([p['ct2_w'][:, :, 0], p['ct2_w'][:, :, 2]], axis=0),
         jnp.concatenate([p['ct2_w'][:, :, 1], p['ct2_w'][:, :, 3]], axis=0)],
        axis=1)
    sd2 = p['dbn2_g'] * inv
    td2 = p['ct2_b'] * sd2 + p['dbn2_b']
    o2 = _mm(x2p.reshape(32 * B, 2 * H), w_ct2,
             jnp.tile(sd2, 2)[None], jnp.tile(td2, 2)[None], act="lrelu",
             half=True)
    d_t = (o2.reshape(32, B, 2, H // 2).transpose(0, 2, 1, 3)
           .reshape(S, B, H // 2))

    w_g1 = p['g1_wih'].T
    cur_rest = jnp.concatenate([d_t.astype(jnp.bfloat16),
                                skel_t.astype(jnp.bfloat16)], axis=-1)
    note_prev = jnp.concatenate(
        [jnp.zeros((1, B, K), f32), note_t[:S - 1]], axis=0)
    u = jnp.stack([jax.random.uniform(jax.random.fold_in(jax.random.key(1), i),
                                      (B, K)) for i in range(S)])
    gate = jnp.logical_and(training != 0, teacher_forcing_ratio >= 1.0)
    gate = gate.astype(f32).reshape(1, 1)
    bf16 = jnp.bfloat16
    rec_t = _dec_scan(cur_rest, note_prev, u, h0_dec,
                      w_g1[K:].astype(bf16), p['g1_bih'][None],
                      w_g1[:K].astype(bf16),
                      p['g1_whh'].T.astype(bf16), p['g1_bhh'][None],
                      p['g2_wih'].T.astype(bf16), p['g2_bih'][None],
                      p['g2_whh'].T.astype(bf16), p['g2_bhh'][None],
                      p['nfc_w'].T, p['nfc_b'][None], gate)
    recon = rec_t.transpose(1, 0, 2)

    loss = loss2.reshape(())
    cmt_loss = cmt2.reshape(())
    perplexity = perp2.reshape(())
    return recon, qz, loss, cmt_loss, enc_idx, perplexity


# speedup vs baseline: 1.4603x; 1.4603x over previous
import functools

import jax
import jax.numpy as jnp
from jax import lax
from jax.experimental import pallas as pl
from jax.experimental.pallas import tpu as pltpu
from jax.experimental.pallas import tpu_sc as plsc

B = 256; S = 64; H = 512; D = 64; NC = 512; SKEL = 256; K = 9
BN_EPS = 1e-5
L4 = S // 4


def _mm_body(x_ref, w_ref, s_ref, t_ref, o_ref, *, act):
    acc = jnp.dot(x_ref[...], w_ref[...], preferred_element_type=jnp.float32)
    y = acc * s_ref[...] + t_ref[...]
    if act == "lrelu":
        y = jnp.where(y >= 0, y, 0.2 * y)
    o_ref[...] = y


def _mm(x, w, scale, shift, act="none", mb=256, half=False):
    if half:
        x = x.astype(jnp.bfloat16)
        w = w.astype(jnp.bfloat16)
    M, Kd = x.shape
    N = w.shape[1]
    return pl.pallas_call(
        functools.partial(_mm_body, act=act),
        grid=(M // mb,),
        in_specs=[
            pl.BlockSpec((mb, Kd), lambda i: (i, 0)),
            pl.BlockSpec((Kd, N), lambda i: (0, 0)),
            pl.BlockSpec((1, N), lambda i: (0, 0)),
            pl.BlockSpec((1, N), lambda i: (0, 0)),
        ],
        out_specs=pl.BlockSpec((mb, N), lambda i: (i, 0)),
        out_shape=jax.ShapeDtypeStruct((M, N), jnp.float32),
    )(x, w, scale, shift)


def _convf_body(*refs, nsrc, l_in):
    srcs = refs[:4 * nsrc]
    w_ref, s_ref, t_ref, o_ref = refs[4 * nsrc:]
    t = pl.program_id(0)
    parts = []
    for j in range(4):
        if j == 0:
            m = (t > 0).astype(jnp.float32)
        elif j == 3:
            m = (2 * t + 2 < l_in).astype(jnp.float32)
        else:
            m = None
        for si in range(nsrc):
            xj = srcs[j * nsrc + si][0]
            parts.append(xj if m is None else xj * m)
    xcat = jnp.concatenate(parts, axis=-1)
    acc = jnp.dot(xcat, w_ref[...], preferred_element_type=jnp.float32)
    y = acc * s_ref[...] + t_ref[...]
    o_ref[0] = jnp.where(y >= 0, y, 0.2 * y)


def _conv_fused(srcs, w, scale, shift):
    nsrc = len(srcs)
    l_in = srcs[0].shape[0]
    l_out = l_in // 2
    n = w.shape[1]
    in_specs = []
    args = []
    for j in range(4):
        for s in srcs:
            c = s.shape[2]
            in_specs.append(pl.BlockSpec(
                (1, B, c),
                lambda t, j=j: (jnp.clip(2 * t + j - 1, 0, l_in - 1), 0, 0)))
            args.append(s)
    zmap = lambda t: (0, 0)
    in_specs += [pl.BlockSpec(w.shape, zmap),
                 pl.BlockSpec((1, n), zmap), pl.BlockSpec((1, n), zmap)]
    return pl.pallas_call(
        functools.partial(_convf_body, nsrc=nsrc, l_in=l_in),
        grid=(l_out,),
        in_specs=in_specs,
        out_specs=pl.BlockSpec((1, B, n), lambda t: (t, 0, 0)),
        out_shape=jax.ShapeDtypeStruct((l_out, B, n), jnp.float32),
    )(*args, w, scale, shift)


def _enc_scan_body(gi_ref, h0_ref, whhT_ref, bhh_ref,
                   so_ref, to_ref, ys_ref, h_s):
    t = pl.program_id(0)

    @pl.when(t == 0)
    def _():
        h_s[...] = h0_ref[...]

    h = h_s[...]
    gh = jnp.dot(h, whhT_ref[...], preferred_element_type=jnp.float32)
    gh = gh + bhh_ref[...]
    gi = gi_ref[0]
    r = jax.nn.sigmoid(gi[:, :H] + gh[:, :H])
    z = jax.nn.sigmoid(gi[:, H:2 * H] + gh[:, H:2 * H])
    n = jnp.tanh(gi[:, 2 * H:] + r * gh[:, 2 * H:])
    h2 = (1.0 - z) * n + z * h
    h_s[...] = h2
    ys_ref[0] = so_ref[...] * h2 + to_ref[...]


def _enc_scan(gi, h0, whhT, bhh, so, to, reverse):
    if reverse:
        tmap = lambda t: (S - 1 - t, 0, 0)
    else:
        tmap = lambda t: (t, 0, 0)
    zmap2 = lambda t: (0, 0)
    return pl.pallas_call(
        _enc_scan_body,
        grid=(S,),
        in_specs=[
            pl.BlockSpec((1, B, 3 * H), tmap),
            pl.BlockSpec((B, H), zmap2),
            pl.BlockSpec((H, 3 * H), zmap2),
            pl.BlockSpec((1, 3 * H), zmap2),
            pl.BlockSpec((1, H), zmap2),
            pl.BlockSpec((1, H), zmap2),
        ],
        out_specs=pl.BlockSpec((1, B, H), tmap),
        out_shape=jax.ShapeDtypeStruct((S, B, H), jnp.float32),
        scratch_shapes=[pltpu.VMEM((B, H), jnp.float32)],
    )(gi, h0, whhT, bhh, so, to)


def _dec_scan_body(cur_ref, np_ref, u_ref, h0_ref, wrest_ref, bih1_ref,
                   woutT_ref,
                   g1whhT_ref, bhh1_ref, g2wihT_ref, bih2_ref, g2whhT_ref,
                   bhh2_ref, nfcT_ref, nfcb_ref, gate_ref,
                   rec_ref, hx0_s, hx1_s, outb_s):
    t = pl.program_id(0)

    @pl.when(t == 0)
    def _():
        hx0_s[...] = h0_ref[...]
        outb_s[...] = jnp.zeros_like(outb_s)

    gate = gate_ref[0, 0] > 0.0
    out = jnp.where(gate, np_ref[0], outb_s[...])
    gi1 = jnp.dot(cur_ref[0], wrest_ref[...],
                  preferred_element_type=jnp.float32) + bih1_ref[...]
    gi1 = gi1 + jnp.dot(out.astype(jnp.bfloat16), woutT_ref[...],
                        preferred_element_type=jnp.float32)
    h0p = hx0_s[...]
    gh1 = jnp.dot(h0p.astype(jnp.bfloat16), g1whhT_ref[...],
                  preferred_element_type=jnp.float32)
    gh1 = gh1 + bhh1_ref[...]
    r1 = jax.nn.sigmoid(gi1[:, :H] + gh1[:, :H])
    z1 = jax.nn.sigmoid(gi1[:, H:2 * H] + gh1[:, H:2 * H])
    n1 = jnp.tanh(gi1[:, 2 * H:] + r1 * gh1[:, 2 * H:])
    hx0n = (1.0 - z1) * n1 + z1 * h0p

    h1p = jnp.where(t == 0, hx0n, hx1_s[...])
    gi2 = jnp.dot(hx0n.astype(jnp.bfloat16), g2wihT_ref[...],
                  preferred_element_type=jnp.float32)
    gi2 = gi2 + bih2_ref[...]
    gh2 = jnp.dot(h1p.astype(jnp.bfloat16), g2whhT_ref[...],
                  preferred_element_type=jnp.float32)
    gh2 = gh2 + bhh2_ref[...]
    r2 = jax.nn.sigmoid(gi2[:, :H] + gh2[:, :H])
    z2 = jax.nn.sigmoid(gi2[:, H:2 * H] + gh2[:, H:2 * H])
    n2 = jnp.tanh(gi2[:, 2 * H:] + r2 * gh2[:, 2 * H:])
    hx1n = (1.0 - z2) * n2 + z2 * h1p

    no = jnp.dot(hx1n, nfcT_ref[...], preferred_element_type=jnp.float32)
    no = no + nfcb_ref[...]
    rec_ref[0] = no
    outb_s[...] = (jax.nn.sigmoid(no) - u_ref[0] > 0).astype(jnp.float32)
    hx0_s[...] = hx0n
    hx1_s[...] = hx1n


def _dec_scan(cur_rest, note_prev, u, h0, wrest, bih1, woutT, g1whhT, bhh1,
              g2wihT, bih2, g2whhT, bhh2, nfcT, nfcb, gate):
    tmap3 = lambda t: (t, 0, 0)
    zmap2 = lambda t: (0, 0)
    cw = cur_rest.shape[2]
    return pl.pallas_call(
        _dec_scan_body,
        grid=(S,),
        in_specs=[
            pl.BlockSpec((1, B, cw), tmap3),
            pl.BlockSpec((1, B, K), tmap3),
            pl.BlockSpec((1, B, K), tmap3),
            pl.BlockSpec((B, H), zmap2),
            pl.BlockSpec((cw, 3 * H), zmap2),
            pl.BlockSpec((1, 3 * H), zmap2),
            pl.BlockSpec((K, 3 * H), zmap2),
            pl.BlockSpec((H, 3 * H), zmap2),
            pl.BlockSpec((1, 3 * H), zmap2),
            pl.BlockSpec((H, 3 * H), zmap2),
            pl.BlockSpec((1, 3 * H), zmap2),
            pl.BlockSpec((H, 3 * H), zmap2),
            pl.BlockSpec((1, 3 * H), zmap2),
            pl.BlockSpec((H, K), zmap2),
            pl.BlockSpec((1, K), zmap2),
            pl.BlockSpec(memory_space=pltpu.SMEM),
        ],
        out_specs=pl.BlockSpec((1, B, K), tmap3),
        out_shape=jax.ShapeDtypeStruct((S, B, K), jnp.float32),
        scratch_shapes=[pltpu.VMEM((B, H), jnp.float32),
                        pltpu.VMEM((B, H), jnp.float32),
                        pltpu.VMEM((B, K), jnp.float32)],
    )(cur_rest, note_prev, u, h0, wrest, bih1, woutT, g1whhT, bhh1, g2wihT,
      bih2, g2whhT, bhh2, nfcT, nfcb, gate)


def _vq_argmin_body(f_ref, et_ref, idx_ref):
    f = f_ref[...]
    et = et_ref[...]
    f2 = jnp.sum(f * f, axis=1, keepdims=True)
    e2 = jnp.sum(et * et, axis=0, keepdims=True)
    dist = f2 - 2.0 * jnp.dot(f, et, preferred_element_type=jnp.float32) + e2
    idx_ref[...] = jnp.argmin(dist, axis=1).astype(jnp.int32)[:, None]


def _vq_argmin(flat, embT):
    M = flat.shape[0]
    return pl.pallas_call(
        _vq_argmin_body,
        in_specs=[pl.BlockSpec((M, D), lambda: (0, 0)),
                  pl.BlockSpec((D, NC), lambda: (0, 0))],
        out_specs=pl.BlockSpec((M, 1), lambda: (0, 0)),
        out_shape=jax.ShapeDtypeStruct((M, 1), jnp.int32),
    )(flat, embT)


def _sc_quant_gather(embed, idx):
    M = idx.shape[0]
    dp = 128
    table = jnp.pad(embed, ((0, 0), (0, dp - D)))
    info = plsc.get_sparse_core_info()
    nw = info.num_cores * info.num_subcores
    bpw = M // nw
    mesh = plsc.VectorSubcoreMesh(core_axis_name="c", subcore_axis_name="s")

    @functools.partial(
        pl.kernel, mesh=mesh,
        out_type=jax.ShapeDtypeStruct((M, dp), jnp.float32),
        scratch_types=[pltpu.VMEM((bpw,), jnp.int32),
                       pltpu.VMEM((bpw, dp), jnp.float32),
                       pltpu.SemaphoreType.DMA],
    )
    def gather_k(table_hbm, idx_hbm, out_hbm, idx_v, rows_v, sem):
        wid = lax.axis_index("s") * info.num_cores + lax.axis_index("c")
        base = wid * bpw
        pltpu.sync_copy(idx_hbm.at[pl.ds(base, bpw)], idx_v)
        pltpu.async_copy(table_hbm.at[idx_v], rows_v, sem).wait()
        pltpu.sync_copy(rows_v, out_hbm.at[pl.ds(base, bpw)])

    return gather_k(table, idx)[:, :D]


def _vq_mse_body(f_ref, q_ref, loss_ref, cmt_ref):
    dq = q_ref[...] - f_ref[...]
    mse = jnp.mean(dq * dq)
    cmt_ref[...] = mse.reshape(1, 1)
    loss_ref[...] = (mse + 0.2 * mse).reshape(1, 1)


def _vq_mse(flat, quant):
    M = flat.shape[0]
    zmap = lambda: (0, 0)
    return pl.pallas_call(
        _vq_mse_body,
        in_specs=[pl.BlockSpec((M, D), zmap),
                  pl.BlockSpec((M, D), zmap)],
        out_specs=[pl.BlockSpec((1, 1), zmap)] * 2,
        out_shape=[jax.ShapeDtypeStruct((1, 1), jnp.float32)] * 2,
    )(flat, quant)


def _vq_perp_body(idx_ref, perp_ref):
    M = idx_ref.shape[0]
    onehot = (idx_ref[...] ==
              lax.broadcasted_iota(jnp.int32, (M, NC), 1)).astype(jnp.float32)
    avg = jnp.sum(onehot, axis=0) * (1.0 / M)
    perp = jnp.exp(-jnp.sum(avg * jnp.log(avg + 1e-10)))
    perp_ref[...] = perp.reshape(1, 1)


def _vq_perp(idx):
    M = idx.shape[0]
    zmap = lambda: (0, 0)
    return pl.pallas_call(
        _vq_perp_body,
        in_specs=[pl.BlockSpec((M, 1), zmap)],
        out_specs=pl.BlockSpec((1, 1), zmap),
        out_shape=jax.ShapeDtypeStruct((1, 1), jnp.float32),
    )(idx)


def kernel(note, skel, skel_encoded, training, teacher_forcing_ratio, params):
    p = params
    f32 = jnp.float32
    inv = 1.0 / jnp.sqrt(jnp.asarray(1.0 + BN_EPS, f32))
    ones_h = jnp.ones((1, H), f32)

    note_t = note.transpose(1, 0, 2)
    skel_t = skel.transpose(1, 0, 2)

    x = _mm(note_t.reshape(S * B, K), p['enc_in_w'].T,
            jnp.ones((1, H), f32), p['enc_in_b'][None])
    gi_f = _mm(x, p['enc_gru_wih_f'].T, jnp.ones((1, 3 * H), f32),
               p['enc_gru_bih_f'][None]).reshape(S, B, 3 * H)
    gi_b = _mm(x, p['enc_gru_wih_b'].T, jnp.ones((1, 3 * H), f32),
               p['enc_gru_bih_b'][None]).reshape(S, B, 3 * H)

    w_hid = jnp.concatenate([p['enc_hfc_w'].T, p['dec_hfc_w'].T], axis=1)
    b_hid = jnp.concatenate([p['enc_hfc_b'], p['dec_hfc_b']])[None]
    hid_all = _mm(skel_encoded, w_hid, jnp.ones((1, 2 * H), f32), b_hid)
    h0_enc = hid_all[:, :H]
    h0_dec = hid_all[:, H:]

    sa = (p['aux1_g'] * inv)[None]
    ta = p['aux1_b'][None]
    yf = _enc_scan(gi_f, h0_enc, p['enc_gru_whh_f'].T,
                   p['enc_gru_bhh_f'][None], sa[:, :H], ta[:, :H], False)
    yb = _enc_scan(gi_b, h0_enc, p['enc_gru_whh_b'].T,
                   p['enc_gru_bhh_b'][None], sa[:, H:], ta[:, H:], True)

    w1 = p['c1_w'].transpose(2, 1, 0).reshape(8 * H, H)
    s1 = (p['bn1_g'] * inv)[None]
    t1 = (p['c1_b'] * s1[0] + p['bn1_b'])[None]
    h1 = _conv_fused([yf, yb], w1, s1, t1)

    w2 = p['c2_w'].transpose(2, 1, 0).reshape(4 * H, H)
    s2 = (p['bn2_g'] * inv)[None]
    t2 = (p['c2_b'] * s2[0] + p['bn2_b'])[None]
    h2 = _conv_fused([h1], w2, s2, t2)

    flat = _mm(h2.reshape(L4 * B, H), p['c3_w'][:, :, 0].T,
               jnp.ones((1, D), f32),
               p['c3_b'][None])

    idx2 = _vq_argmin(flat, p['embed'].T)
    idx = idx2.reshape(L4 * B)
    quant = _sc_quant_gather(p['embed'], idx)
    perp2 = _vq_perp(idx2)
    loss2, cmt2 = _vq_mse(flat, quant)

    q_sb = quant.reshape(L4, B, D)
    qz = q_sb.transpose(1, 2, 0)
    enc_idx = idx.reshape(L4, B).T[:, :, None]

    qp = jnp.pad(q_sb.astype(jnp.bfloat16), ((1, 2), (0, 0), (0, 0)))
    x1p = jnp.concatenate([qp[1:19], qp[0:18]], axis=-1)
    w_ct1 = jnp.concatenate(
        [jnp.concatenate([p['ct1_w'][:, :, 0], p['ct1_w'][:, :, 2]], axis=0),
         jnp.concatenate([p['ct1_w'][:, :, 1], p['ct1_w'][:, :, 3]], axis=0)],
        axis=1)
    sd1 = p['dbn1_g'] * inv
    td1 = p['ct1_b'] * sd1 + p['dbn1_b']
    o1 = _mm(x1p.reshape(18 * B, 2 * D), w_ct1,
             jnp.tile(sd1, 2)[None], jnp.tile(td1, 2)[None], act="lrelu",
             half=True)
    d1 = (o1.reshape(18, B, 2, H).transpose(0, 2, 1, 3)
          .reshape(36, B, H)[:35])

    d1p = jnp.pad(d1.astype(jnp.bfloat16), ((1, 0), (0, 0), (0, 0)))
    x2p = jnp.concatenate([d1p[1:33], d1p[0:32]], axis=-1)
    w_ct2 = jnp.concatenate(
        [jnp.concatenate([p['ct2_w'][:, :, 0], p['ct2_w'][:, :, 2]], axis=0),
         jnp.concatenate([p['ct2_w'][:, :, 1], p['ct2_w'][:, :, 3]], axis=0)],
        axis=1)
    sd2 = p['dbn2_g'] * inv
    td2 = p['ct2_b'] * sd2 + p['dbn2_b']
    o2 = _mm(x2p.reshape(32 * B, 2 * H), w_ct2,
             jnp.tile(sd2, 2)[None], jnp.tile(td2, 2)[None], act="lrelu",
             half=True)
    d_t = (o2.reshape(32, B, 2, H // 2).transpose(0, 2, 1, 3)
           .reshape(S, B, H // 2))

    w_g1 = p['g1_wih'].T
    cur_rest = jnp.concatenate([d_t.astype(jnp.bfloat16),
                                skel_t.astype(jnp.bfloat16)], axis=-1)
    note_prev = jnp.concatenate(
        [jnp.zeros((1, B, K), f32), note_t[:S - 1]], axis=0)
    u = jax.vmap(lambda i: jax.random.uniform(
        jax.random.fold_in(jax.random.key(1), i), (B, K)))(jnp.arange(S))
    gate = jnp.logical_and(training != 0, teacher_forcing_ratio >= 1.0)
    gate = gate.astype(f32).reshape(1, 1)
    bf16 = jnp.bfloat16
    rec_t = _dec_scan(cur_rest, note_prev, u, h0_dec,
                      w_g1[K:].astype(bf16), p['g1_bih'][None],
                      w_g1[:K].astype(bf16),
                      p['g1_whh'].T.astype(bf16), p['g1_bhh'][None],
                      p['g2_wih'].T.astype(bf16), p['g2_bih'][None],
                      p['g2_whh'].T.astype(bf16), p['g2_bhh'][None],
                      p['nfc_w'].T, p['nfc_b'][None], gate)
    recon = rec_t.transpose(1, 0, 2)

    loss = loss2.reshape(())
    cmt_loss = cmt2.reshape(())
    perplexity = perp2.reshape(())
    return recon, qz, loss, cmt_loss, enc_idx, perplexity


# speedup vs baseline: 1.4617x; 1.0009x over previous
import functools

import jax
import jax.numpy as jnp
from jax import lax
from jax.experimental import pallas as pl
from jax.experimental.pallas import tpu as pltpu
from jax.experimental.pallas import tpu_sc as plsc

B = 256; S = 64; H = 512; D = 64; NC = 512; SKEL = 256; K = 9
BN_EPS = 1e-5
L4 = S // 4


def _mm_body(x_ref, w_ref, s_ref, t_ref, o_ref, *, act):
    acc = jnp.dot(x_ref[...], w_ref[...], preferred_element_type=jnp.float32)
    y = acc * s_ref[...] + t_ref[...]
    if act == "lrelu":
        y = jnp.where(y >= 0, y, 0.2 * y)
    o_ref[...] = y


def _mm(x, w, scale, shift, act="none", mb=256, half=False):
    if half:
        x = x.astype(jnp.bfloat16)
        w = w.astype(jnp.bfloat16)
    M, Kd = x.shape
    N = w.shape[1]
    return pl.pallas_call(
        functools.partial(_mm_body, act=act),
        grid=(M // mb,),
        in_specs=[
            pl.BlockSpec((mb, Kd), lambda i: (i, 0)),
            pl.BlockSpec((Kd, N), lambda i: (0, 0)),
            pl.BlockSpec((1, N), lambda i: (0, 0)),
            pl.BlockSpec((1, N), lambda i: (0, 0)),
        ],
        out_specs=pl.BlockSpec((mb, N), lambda i: (i, 0)),
        out_shape=jax.ShapeDtypeStruct((M, N), jnp.float32),
    )(x, w, scale, shift)


def _convf_body(*refs, nsrc, l_in):
    srcs = refs[:4 * nsrc]
    w_ref, s_ref, t_ref, o_ref = refs[4 * nsrc:]
    t = pl.program_id(0)
    parts = []
    for j in range(4):
        if j == 0:
            m = (t > 0).astype(jnp.float32)
        elif j == 3:
            m = (2 * t + 2 < l_in).astype(jnp.float32)
        else:
            m = None
        for si in range(nsrc):
            xj = srcs[j * nsrc + si][0]
            parts.append(xj if m is None else xj * m)
    xcat = jnp.concatenate(parts, axis=-1)
    acc = jnp.dot(xcat, w_ref[...], preferred_element_type=jnp.float32)
    y = acc * s_ref[...] + t_ref[...]
    o_ref[0] = jnp.where(y >= 0, y, 0.2 * y)


def _conv_fused(srcs, w, scale, shift):
    nsrc = len(srcs)
    l_in = srcs[0].shape[0]
    l_out = l_in // 2
    n = w.shape[1]
    in_specs = []
    args = []
    for j in range(4):
        for s in srcs:
            c = s.shape[2]
            in_specs.append(pl.BlockSpec(
                (1, B, c),
                lambda t, j=j: (jnp.clip(2 * t + j - 1, 0, l_in - 1), 0, 0)))
            args.append(s)
    zmap = lambda t: (0, 0)
    in_specs += [pl.BlockSpec(w.shape, zmap),
                 pl.BlockSpec((1, n), zmap), pl.BlockSpec((1, n), zmap)]
    return pl.pallas_call(
        functools.partial(_convf_body, nsrc=nsrc, l_in=l_in),
        grid=(l_out,),
        in_specs=in_specs,
        out_specs=pl.BlockSpec((1, B, n), lambda t: (t, 0, 0)),
        out_shape=jax.ShapeDtypeStruct((l_out, B, n), jnp.float32),
    )(*args, w, scale, shift)


def _enc_scan_body(gi_ref, h0_ref, whhT_ref, bhh_ref,
                   so_ref, to_ref, ys_ref, h_s):
    t = pl.program_id(0)

    @pl.when(t == 0)
    def _():
        h_s[...] = h0_ref[...]

    h = h_s[...]
    gh = jnp.dot(h, whhT_ref[...], preferred_element_type=jnp.float32)
    gh = gh + bhh_ref[...]
    gi = gi_ref[0]
    r = jax.nn.sigmoid(gi[:, :H] + gh[:, :H])
    z = jax.nn.sigmoid(gi[:, H:2 * H] + gh[:, H:2 * H])
    n = jnp.tanh(gi[:, 2 * H:] + r * gh[:, 2 * H:])
    h2 = (1.0 - z) * n + z * h
    h_s[...] = h2
    ys_ref[0] = so_ref[...] * h2 + to_ref[...]


def _enc_scan(gi, h0, whhT, bhh, so, to, reverse):
    if reverse:
        tmap = lambda t: (S - 1 - t, 0, 0)
    else:
        tmap = lambda t: (t, 0, 0)
    zmap2 = lambda t: (0, 0)
    return pl.pallas_call(
        _enc_scan_body,
        grid=(S,),
        in_specs=[
            pl.BlockSpec((1, B, 3 * H), tmap),
            pl.BlockSpec((B, H), zmap2),
            pl.BlockSpec((H, 3 * H), zmap2),
            pl.BlockSpec((1, 3 * H), zmap2),
            pl.BlockSpec((1, H), zmap2),
            pl.BlockSpec((1, H), zmap2),
        ],
        out_specs=pl.BlockSpec((1, B, H), tmap),
        out_shape=jax.ShapeDtypeStruct((S, B, H), jnp.float32),
        scratch_shapes=[pltpu.VMEM((B, H), jnp.float32)],
    )(gi, h0, whhT, bhh, so, to)


def _dec_scan_body(cur_ref, np_ref, u_ref, h0_ref, wrest_ref, bih1_ref,
                   woutT_ref,
                   g1whhT_ref, bhh1_ref, g2wihT_ref, bih2_ref, g2whhT_ref,
                   bhh2_ref, nfcT_ref, nfcb_ref, gate_ref,
                   rec_ref, hx0_s, hx1_s, outb_s):
    t = pl.program_id(0)

    @pl.when(t == 0)
    def _():
        hx0_s[...] = h0_ref[...]
        outb_s[...] = jnp.zeros_like(outb_s)

    gate = gate_ref[0, 0] > 0.0
    out = jnp.where(gate, np_ref[0], outb_s[...])
    gi1 = jnp.dot(cur_ref[0], wrest_ref[...],
                  preferred_element_type=jnp.float32) + bih1_ref[...]
    gi1 = gi1 + jnp.dot(out.astype(jnp.bfloat16), woutT_ref[...],
                        preferred_element_type=jnp.float32)
    h0p = hx0_s[...]
    gh1 = jnp.dot(h0p.astype(jnp.bfloat16), g1whhT_ref[...],
                  preferred_element_type=jnp.float32)
    gh1 = gh1 + bhh1_ref[...]
    r1 = jax.nn.sigmoid(gi1[:, :H] + gh1[:, :H])
    z1 = jax.nn.sigmoid(gi1[:, H:2 * H] + gh1[:, H:2 * H])
    n1 = jnp.tanh(gi1[:, 2 * H:] + r1 * gh1[:, 2 * H:])
    hx0n = (1.0 - z1) * n1 + z1 * h0p

    h1p = jnp.where(t == 0, hx0n, hx1_s[...])
    gi2 = jnp.dot(hx0n.astype(jnp.bfloat16), g2wihT_ref[...],
                  preferred_element_type=jnp.float32)
    gi2 = gi2 + bih2_ref[...]
    gh2 = jnp.dot(h1p.astype(jnp.bfloat16), g2whhT_ref[...],
                  preferred_element_type=jnp.float32)
    gh2 = gh2 + bhh2_ref[...]
    r2 = jax.nn.sigmoid(gi2[:, :H] + gh2[:, :H])
    z2 = jax.nn.sigmoid(gi2[:, H:2 * H] + gh2[:, H:2 * H])
    n2 = jnp.tanh(gi2[:, 2 * H:] + r2 * gh2[:, 2 * H:])
    hx1n = (1.0 - z2) * n2 + z2 * h1p

    no = jnp.dot(hx1n, nfcT_ref[...], preferred_element_type=jnp.float32)
    no = no + nfcb_ref[...]
    rec_ref[0] = no
    outb_s[...] = (jax.nn.sigmoid(no) - u_ref[0] > 0).astype(jnp.float32)
    hx0_s[...] = hx0n
    hx1_s[...] = hx1n


def _dec_scan(cur_rest, note_prev, u, h0, wrest, bih1, woutT, g1whhT, bhh1,
              g2wihT, bih2, g2whhT, bhh2, nfcT, nfcb, gate):
    tmap3 = lambda t: (t, 0, 0)
    zmap2 = lambda t: (0, 0)
    cw = cur_rest.shape[2]
    return pl.pallas_call(
        _dec_scan_body,
        grid=(S,),
        in_specs=[
            pl.BlockSpec((1, B, cw), tmap3),
            pl.BlockSpec((1, B, K), tmap3),
            pl.BlockSpec((1, B, K), tmap3),
            pl.BlockSpec((B, H), zmap2),
            pl.BlockSpec((cw, 3 * H), zmap2),
            pl.BlockSpec((1, 3 * H), zmap2),
            pl.BlockSpec((K, 3 * H), zmap2),
            pl.BlockSpec((H, 3 * H), zmap2),
            pl.BlockSpec((1, 3 * H), zmap2),
            pl.BlockSpec((H, 3 * H), zmap2),
            pl.BlockSpec((1, 3 * H), zmap2),
            pl.BlockSpec((H, 3 * H), zmap2),
            pl.BlockSpec((1, 3 * H), zmap2),
            pl.BlockSpec((H, K), zmap2),
            pl.BlockSpec((1, K), zmap2),
            pl.BlockSpec(memory_space=pltpu.SMEM),
        ],
        out_specs=pl.BlockSpec((1, B, K), tmap3),
        out_shape=jax.ShapeDtypeStruct((S, B, K), jnp.float32),
        scratch_shapes=[pltpu.VMEM((B, H), jnp.float32),
                        pltpu.VMEM((B, H), jnp.float32),
                        pltpu.VMEM((B, K), jnp.float32)],
    )(cur_rest, note_prev, u, h0, wrest, bih1, woutT, g1whhT, bhh1, g2wihT,
      bih2, g2whhT, bhh2, nfcT, nfcb, gate)


def _vq_argmin_body(f_ref, et_ref, idx_ref):
    f = f_ref[...]
    et = et_ref[...]
    f2 = jnp.sum(f * f, axis=1, keepdims=True)
    e2 = jnp.sum(et * et, axis=0, keepdims=True)
    dist = f2 - 2.0 * jnp.dot(f, et, preferred_element_type=jnp.float32) + e2
    idx_ref[...] = jnp.argmin(dist, axis=1).astype(jnp.int32)[:, None]


def _vq_argmin(flat, embT):
    M = flat.shape[0]
    return pl.pallas_call(
        _vq_argmin_body,
        in_specs=[pl.BlockSpec((M, D), lambda: (0, 0)),
                  pl.BlockSpec((D, NC), lambda: (0, 0))],
        out_specs=pl.BlockSpec((M, 1), lambda: (0, 0)),
        out_shape=jax.ShapeDtypeStruct((M, 1), jnp.int32),
    )(flat, embT)


def _sc_quant_gather(embed, idx):
    M = idx.shape[0]
    dp = 128
    table = jnp.pad(embed, ((0, 0), (0, dp - D)))
    info = plsc.get_sparse_core_info()
    nw = info.num_cores * info.num_subcores
    bpw = M // nw
    mesh = plsc.VectorSubcoreMesh(core_axis_name="c", subcore_axis_name="s")

    @functools.partial(
        pl.kernel, mesh=mesh,
        out_type=jax.ShapeDtypeStruct((M, dp), jnp.float32),
        scratch_types=[pltpu.VMEM((bpw,), jnp.int32),
                       pltpu.VMEM((bpw, dp), jnp.float32),
                       pltpu.SemaphoreType.DMA],
    )
    def gather_k(table_hbm, idx_hbm, out_hbm, idx_v, rows_v, sem):
        wid = lax.axis_index("s") * info.num_cores + lax.axis_index("c")
        base = wid * bpw
        pltpu.sync_copy(idx_hbm.at[pl.ds(base, bpw)], idx_v)
        pltpu.async_copy(table_hbm.at[idx_v], rows_v, sem).wait()
        pltpu.sync_copy(rows_v, out_hbm.at[pl.ds(base, bpw)])

    return gather_k(table, idx)[:, :D]


def _vq_mse_body(f_ref, q_ref, loss_ref, cmt_ref):
    dq = q_ref[...] - f_ref[...]
    mse = jnp.mean(dq * dq)
    cmt_ref[...] = mse.reshape(1, 1)
    loss_ref[...] = (mse + 0.2 * mse).reshape(1, 1)


def _vq_mse(flat, quant):
    M = flat.shape[0]
    zmap = lambda: (0, 0)
    return pl.pallas_call(
        _vq_mse_body,
        in_specs=[pl.BlockSpec((M, D), zmap),
                  pl.BlockSpec((M, D), zmap)],
        out_specs=[pl.BlockSpec((1, 1), zmap)] * 2,
        out_shape=[jax.ShapeDtypeStruct((1, 1), jnp.float32)] * 2,
    )(flat, quant)


def _vq_perp_body(idx_ref, perp_ref):
    M = idx_ref.shape[0]
    onehot = (idx_ref[...] ==
              lax.broadcasted_iota(jnp.int32, (M, NC), 1)).astype(jnp.float32)
    avg = jnp.sum(onehot, axis=0) * (1.0 / M)
    perp = jnp.exp(-jnp.sum(avg * jnp.log(avg + 1e-10)))
    perp_ref[...] = perp.reshape(1, 1)


def _vq_perp(idx):
    M = idx.shape[0]
    zmap = lambda: (0, 0)
    return pl.pallas_call(
        _vq_perp_body,
        in_specs=[pl.BlockSpec((M, 1), zmap)],
        out_specs=pl.BlockSpec((1, 1), zmap),
        out_shape=jax.ShapeDtypeStruct((1, 1), jnp.float32),
    )(idx)


def kernel(note, skel, skel_encoded, training, teacher_forcing_ratio, params):
    p = params
    f32 = jnp.float32
    inv = 1.0 / jnp.sqrt(jnp.asarray(1.0 + BN_EPS, f32))

    note_t = note.transpose(1, 0, 2)
    skel_t = skel.transpose(1, 0, 2)

    x = _mm(note_t.reshape(S * B, K), p['enc_in_w'].T,
            jnp.ones((1, H), f32), p['enc_in_b'][None])
    gi_f = _mm(x, p['enc_gru_wih_f'].T, jnp.ones((1, 3 * H), f32),
               p['enc_gru_bih_f'][None]).reshape(S, B, 3 * H)
    gi_b = _mm(x, p['enc_gru_wih_b'].T, jnp.ones((1, 3 * H), f32),
               p['enc_gru_bih_b'][None]).reshape(S, B, 3 * H)

    w_hid = jnp.concatenate([p['enc_hfc_w'].T, p['dec_hfc_w'].T], axis=1)
    b_hid = jnp.concatenate([p['enc_hfc_b'], p['dec_hfc_b']])[None]
    hid_all = _mm(skel_encoded, w_hid, jnp.ones((1, 2 * H), f32), b_hid)
    h0_enc = hid_all[:, :H]
    h0_dec = hid_all[:, H:]

    sa = (p['aux1_g'] * inv)[None]
    ta = p['aux1_b'][None]
    yf = _enc_scan(gi_f, h0_enc, p['enc_gru_whh_f'].T,
                   p['enc_gru_bhh_f'][None], sa[:, :H], ta[:, :H], False)
    yb = _enc_scan(gi_b, h0_enc, p['enc_gru_whh_b'].T,
                   p['enc_gru_bhh_b'][None], sa[:, H:], ta[:, H:], True)

    w1 = p['c1_w'].transpose(2, 1, 0).reshape(8 * H, H)
    s1 = (p['bn1_g'] * inv)[None]
    t1 = (p['c1_b'] * s1[0] + p['bn1_b'])[None]
    h1 = _conv_fused([yf, yb], w1, s1, t1)

    w2 = p['c2_w'].transpose(2, 1, 0).reshape(4 * H, H)
    s2 = (p['bn2_g'] * inv)[None]
    t2 = (p['c2_b'] * s2[0] + p['bn2_b'])[None]
    h2 = _conv_fused([h1], w2, s2, t2)

    flat = _mm(h2.reshape(L4 * B, H), p['c3_w'][:, :, 0].T,
               jnp.ones((1, D), f32),
               p['c3_b'][None])

    idx2 = _vq_argmin(flat, p['embed'].T)
    idx = idx2.reshape(L4 * B)
    quant = _sc_quant_gather(p['embed'], idx)
    perp2 = _vq_perp(idx2)
    loss2, cmt2 = _vq_mse(flat, quant)

    q_sb = quant.reshape(L4, B, D)
    qz = q_sb.transpose(1, 2, 0)
    enc_idx = idx.reshape(L4, B).T[:, :, None]

    qp = jnp.pad(q_sb.astype(jnp.bfloat16), ((1, 2), (0, 0), (0, 0)))
    x1p = jnp.concatenate([qp[1:19], qp[0:18]], axis=-1)
    w_ct1 = jnp.concatenate(
        [jnp.concatenate([p['ct1_w'][:, :, 0], p['ct1_w'][:, :, 2]], axis=0),
         jnp.concatenate([p['ct1_w'][:, :, 1], p['ct1_w'][:, :, 3]], axis=0)],
        axis=1)
    sd1 = p['dbn1_g'] * inv
    td1 = p['ct1_b'] * sd1 + p['dbn1_b']
    o1 = _mm(x1p.reshape(18 * B, 2 * D), w_ct1,
             jnp.tile(sd1, 2)[None], jnp.tile(td1, 2)[None], act="lrelu",
             half=True)
    d1 = (o1.reshape(18, B, 2, H).transpose(0, 2, 1, 3)
          .reshape(36, B, H)[:35])

    d1p = jnp.pad(d1.astype(jnp.bfloat16), ((1, 0), (0, 0), (0, 0)))
    x2p = jnp.concatenate([d1p[1:33], d1p[0:32]], axis=-1)
    w_ct2 = jnp.concatenate(
        [jnp.concatenate([p['ct2_w'][:, :, 0], p['ct2_w'][:, :, 2]], axis=0),
         jnp.concatenate([p['ct2_w'][:, :, 1], p['ct2_w'][:, :, 3]], axis=0)],
        axis=1)
    sd2 = p['dbn2_g'] * inv
    td2 = p['ct2_b'] * sd2 + p['dbn2_b']
    o2 = _mm(x2p.reshape(32 * B, 2 * H), w_ct2,
             jnp.tile(sd2, 2)[None], jnp.tile(td2, 2)[None], act="lrelu",
             half=True)
    d_t = (o2.reshape(32, B, 2, H // 2).transpose(0, 2, 1, 3)
           .reshape(S, B, H // 2))

    w_g1 = p['g1_wih'].T
    cur_rest = jnp.concatenate([d_t.astype(jnp.bfloat16),
                                skel_t.astype(jnp.bfloat16)], axis=-1)
    note_prev = jnp.concatenate(
        [jnp.zeros((1, B, K), f32), note_t[:S - 1]], axis=0)
    u = jax.vmap(lambda i: jax.random.uniform(
        jax.random.fold_in(jax.random.key(1), i), (B, K)))(jnp.arange(S))
    gate = jnp.logical_and(training != 0, teacher_forcing_ratio >= 1.0)
    gate = gate.astype(f32).reshape(1, 1)
    bf16 = jnp.bfloat16
    rec_t = _dec_scan(cur_rest, note_prev, u, h0_dec,
                      w_g1[K:].astype(bf16), p['g1_bih'][None],
                      w_g1[:K].astype(bf16),
                      p['g1_whh'].T.astype(bf16), p['g1_bhh'][None],
                      p['g2_wih'].T.astype(bf16), p['g2_bih'][None],
                      p['g2_whh'].T.astype(bf16), p['g2_bhh'][None],
                      p['nfc_w'].T, p['nfc_b'][None], gate)
    recon = rec_t.transpose(1, 0, 2)

    loss = loss2.reshape(())
    cmt_loss = cmt2.reshape(())
    perplexity = perp2.reshape(())
    return recon, qz, loss, cmt_loss, enc_idx, perplexity
